# Initial kernel scaffold; baseline (speedup 1.0000x reference)
import jax
import jax.numpy as jnp
from jax.experimental import pallas as pl


def kernel(xs, onehots, adjs, W, att_l, att_r, conv1_w, conv1_b, conv2_w, conv2_b, oh_lin_w, oh_lin_b, bias):
    raise NotImplementedError("write your pallas kernel here")


# speedup vs baseline: 2.4683x; 2.4683x over previous
import functools

import jax
import jax.numpy as jnp
from jax import lax
from jax.experimental import pallas as pl
from jax.experimental.pallas import tpu as pltpu
from jax.experimental.pallas import tpu_sc as plsc

N = 10000
E = 160000
IN_CH = 256
OUT_CH = 256
HEADS = 4
OH_DIM = 64
OH_CH = 8
NEG_SLOPE = 0.2

NC = 2
NS = 16
KB = 128
N_PAD = 10240
E_PAD = 163840
N_HALF = 5120
S_ROWS = 5248
INIT_PER_SID = S_ROWS // NS
OUT_PER_SID = N_HALF // NS
ROWS_PER_SID = N_PAD // NS
A_EDGES_PER_W = E_PAD // (NC * NS)
A_BLOCKS = A_EDGES_PER_W // KB
C_EDGES_PER_S = E_PAD // NS
C_BLOCKS = C_EDGES_PER_S // KB
BN = 400
G1 = N // BN


def _symlog(x):
    return jnp.sign(x) * jnp.log1p(jnp.abs(x))


def _dense_prep_body(xs_ref, soh_ref, oh_ref, c1w_ref, c1b_ref, c2w_ref,
                     c2b_ref, olw_ref, olb_ref, w_ref, attl_ref, attr_ref,
                     xmm_ref, ps_ref, pr_ref):
    soh = soh_ref[...]
    prep = _symlog(soh)
    z = jnp.zeros((BN, 1), jnp.float32)

    def lsh(a):
        return jnp.concatenate([a[:, 1:], z], axis=1)

    def rsh(a):
        return jnp.concatenate([z, a[:, :-1]], axis=1)

    h1 = []
    for c in range(8):
        acc = (c1w_ref[c, 0] * rsh(prep) + c1w_ref[c, 1] * prep
               + c1w_ref[c, 2] * lsh(prep) + c1b_ref[0, c])
        h1.append(jnp.maximum(acc, 0.0))
    h1l = [lsh(a) for a in h1]
    h1r = [rsh(a) for a in h1]
    cols = []
    for o in range(16):
        acc = jnp.zeros((BN, OH_DIM), jnp.float32) + c2b_ref[0, o]
        for i in range(8):
            acc = (acc + c2w_ref[o, 3 * i] * h1r[i]
                   + c2w_ref[o, 3 * i + 1] * h1[i]
                   + c2w_ref[o, 3 * i + 2] * h1l[i])
        acc = jnp.maximum(acc, 0.0)
        cols.append(jnp.mean(acc, axis=1).reshape(BN, 1))
    hbar = jnp.concatenate(cols, axis=1)
    poh = jnp.dot(hbar, olw_ref[...],
                  preferred_element_type=jnp.float32) + olb_ref[...]
    xcat = jnp.concatenate([xs_ref[...], poh], axis=1)
    xmm = jnp.dot(xcat, w_ref[...], preferred_element_type=jnp.float32)
    xmm_ref[...] = xmm
    sa_cols = []
    ra_cols = []
    for h in range(HEADS):
        blkh = xmm[:, OUT_CH * h:OUT_CH * (h + 1)]
        sa_cols.append(jnp.sum(blkh * attl_ref[h:h + 1, :],
                               axis=1).reshape(BN, 1))
        ra_cols.append(jnp.sum(blkh * attr_ref[h:h + 1, :],
                               axis=1).reshape(BN, 1))
    zpad12 = jnp.zeros((BN, 12), jnp.float32)
    zpad48 = jnp.zeros((BN, 48), jnp.float32)
    oh = oh_ref[...]
    ps_ref[...] = jnp.concatenate(sa_cols + [zpad12, oh, zpad48], axis=1)
    pr_ref[...] = jnp.concatenate(ra_cols + [zpad12, oh, zpad48], axis=1)


def _dense_prep(xs, soh, oh, c1w, c1b, c2w, c2b, olw, olb, W, attl, attr):
    full0 = lambda *s: pl.BlockSpec(s, lambda i: tuple(0 for _ in s))
    return pl.pallas_call(
        _dense_prep_body,
        grid=(G1,),
        in_specs=[
            pl.BlockSpec((BN, IN_CH), lambda i: (i, 0)),
            pl.BlockSpec((BN, OH_DIM), lambda i: (i, 0)),
            pl.BlockSpec((BN, OH_DIM), lambda i: (i, 0)),
            full0(8, 3), full0(1, 8), full0(16, 24), full0(1, 16),
            full0(16, 8), full0(1, 8),
            full0(IN_CH + OH_CH, HEADS * OUT_CH),
            full0(HEADS, OUT_CH), full0(HEADS, OUT_CH),
        ],
        out_specs=[
            pl.BlockSpec((BN, HEADS * OUT_CH), lambda i: (i, 0)),
            pl.BlockSpec((BN, 128), lambda i: (i, 0)),
            pl.BlockSpec((BN, 128), lambda i: (i, 0)),
        ],
        out_shape=[
            jax.ShapeDtypeStruct((N, HEADS * OUT_CH), jnp.float32),
            jax.ShapeDtypeStruct((N, 128), jnp.float32),
            jax.ShapeDtypeStruct((N, 128), jnp.float32),
        ],
    )(xs, soh, oh, c1w, c1b, c2w, c2b, olw, olb, W, attl, attr)


def _vsum16(x):
    for k in (1, 2, 4, 8):
        idx = jnp.bitwise_xor(lax.iota(jnp.int32, 16), k)
        x = x + x.at[idx].get(mode="promise_in_bounds", unique_indices=True)
    return x


def _zvec16():
    return (lax.iota(jnp.int32, 16) * 0).astype(jnp.float32)


@functools.cache
def _sc_mesh():
    return plsc.VectorSubcoreMesh(core_axis_name="c", subcore_axis_name="s")


def _edge_pass_a_call(ps_pad, pr_pad, snd, rcv):
    wrapped = functools.partial(
        pl.kernel, mesh=_sc_mesh(),
        out_type=[
            jax.ShapeDtypeStruct((E_PAD, 16), jnp.float32),
            jax.ShapeDtypeStruct((NC, N_PAD, 128), jnp.float32),
        ],
        scratch_types=[
            pltpu.VMEM((KB,), jnp.int32),
            pltpu.VMEM((KB,), jnp.int32),
            pltpu.VMEM((1, KB), jnp.int32),
            pltpu.VMEM((KB, 128), jnp.float32),
            pltpu.VMEM((KB, 128), jnp.float32),
            pltpu.VMEM((KB, 16), jnp.float32),
            pltpu.VMEM((8, 128), jnp.float32),
            pltpu.VMEM_SHARED((S_ROWS, 128), jnp.float32),
            pltpu.SemaphoreType.DMA,
        ],
    )
    return wrapped(_edge_pass_a)(ps_pad, pr_pad, snd, rcv)


def _remap_half(ridxs, half):
    base = half * N_HALF
    for j in range(8):
        iv = ridxs[0, pl.ds(16 * j, 16)]
        loc = iv - base
        ok = (loc >= 0) & (loc < N_HALF)
        ridxs[0, pl.ds(16 * j, 16)] = jnp.where(ok, loc, N_HALF)


def _edge_pass_a(ps_hbm, pr_hbm, snd_hbm, rcv_hbm,
                 ex_hbm, part_hbm,
                 sidx, ridxg, ridxs, srows, rrows, exbuf, zb, slab, sem):
    cid = lax.axis_index("c")
    sid = lax.axis_index("s")
    wid = sid * NC + cid
    zv = _zvec16()

    def zrow(r, carry):
        for j in range(8):
            zb[r, pl.ds(16 * j, 16)] = zv
        return carry
    lax.fori_loop(0, 8, zrow, 0)

    base0 = wid * A_EDGES_PER_W

    for half in range(2):
        irow0 = sid * INIT_PER_SID

        def zslab(t, carry):
            pltpu.sync_copy(zb, slab.at[pl.ds(irow0 + 8 * t, 8)])
            return carry
        lax.fori_loop(0, INIT_PER_SID // 8, zslab, 0)
        plsc.subcore_barrier()

        def blk(b, carry):
            base = base0 + b * KB
            pltpu.sync_copy(snd_hbm.at[pl.ds(base, KB)], sidx)
            pltpu.sync_copy(rcv_hbm.at[pl.ds(base, KB)], ridxg)
            pltpu.sync_copy(rcv_hbm.at[pl.ds(base, KB)], ridxs.at[0])
            _remap_half(ridxs, half)
            pltpu.async_copy(ps_hbm.at[sidx], srows, sem).wait()
            pltpu.async_copy(pr_hbm.at[ridxg], rrows, sem).wait()

            def per_edge(e, c2):
                s16 = srows[e, pl.ds(0, 16)]
                r16 = rrows[e, pl.ds(0, 16)]
                a16 = s16 + r16
                a16 = jnp.maximum(a16, a16 * NEG_SLOPE)
                dacc = None
                for j in range(4):
                    so = srows[e, pl.ds(16 + 16 * j, 16)]
                    ro = rrows[e, pl.ds(16 + 16 * j, 16)]
                    dacc = so * ro if dacc is None else dacc + so * ro
                dv = _vsum16(dacc)
                ex16 = jnp.exp(a16) * (1.0 + dv)
                srows[e, pl.ds(0, 16)] = ex16
                exbuf[e, pl.ds(0, 16)] = ex16
                return c2
            lax.fori_loop(0, KB, per_edge, 0)
            if half == 0:
                pltpu.sync_copy(exbuf, ex_hbm.at[pl.ds(base, KB)])
            pltpu.sync_copy(srows, slab.at[ridxs.at[0]], add=True)
            return carry
        lax.fori_loop(0, A_BLOCKS, blk, 0)
        plsc.subcore_barrier()

        orow0 = sid * OUT_PER_SID

        def wout(t, carry):
            r = orow0 + 64 * t
            pltpu.sync_copy(slab.at[pl.ds(r, 64)],
                            part_hbm.at[cid, pl.ds(half * N_HALF + r, 64)])
            return carry
        lax.fori_loop(0, OUT_PER_SID // 64, wout, 0)
        plsc.subcore_barrier()


def _combine_body(p_ref, oh_ref, invd_ref, noh_ref):
    p = p_ref[0] + p_ref[1]
    invd_ref[...] = 1.0 / (p + 1e-16)
    noh_ref[...] = oh_ref[...] + p[:, 16:80]


def _combine(parts, oh_pad):
    BR = 640
    GC = N_PAD // BR
    return pl.pallas_call(
        _combine_body,
        grid=(GC,),
        in_specs=[
            pl.BlockSpec((NC, BR, 128), lambda i: (0, i, 0)),
            pl.BlockSpec((BR, 64), lambda i: (i, 0)),
        ],
        out_specs=[
            pl.BlockSpec((BR, 128), lambda i: (i, 0)),
            pl.BlockSpec((BR, 64), lambda i: (i, 0)),
        ],
        out_shape=[
            jax.ShapeDtypeStruct((N_PAD, 128), jnp.float32),
            jax.ShapeDtypeStruct((N_PAD, 64), jnp.float32),
        ],
    )(parts, oh_pad)


def _edge_pass_c_call(*args):
    wrapped = functools.partial(
        pl.kernel, mesh=_sc_mesh(),
        out_type=[jax.ShapeDtypeStruct((8, N_PAD, 128), jnp.float32)],
        scratch_types=[
            pltpu.VMEM((KB,), jnp.int32),
            pltpu.VMEM((KB,), jnp.int32),
            pltpu.VMEM((1, KB), jnp.int32),
            pltpu.VMEM((KB, 128), jnp.float32),
            pltpu.VMEM((KB, 16), jnp.float32),
            pltpu.VMEM((KB, 128), jnp.float32),
            pltpu.VMEM((8, 128), jnp.float32),
            pltpu.VMEM((1, 128), jnp.float32),
            pltpu.VMEM_SHARED((S_ROWS, 128), jnp.float32),
            pltpu.SemaphoreType.DMA,
        ],
    )
    return wrapped(_edge_pass_c)(*args)


def _edge_pass_c(x0, x1, x2, x3, x4, x5, x6, x7,
                 invd_hbm, ex_hbm, snd_hbm, rcv_hbm, b3_hbm,
                 outc_hbm,
                 sidx, ridxg, ridxs, invrows, exrows, xrows, bbuf, bvec,
                 slab, sem):
    cid = lax.axis_index("c")
    sid = lax.axis_index("s")
    xc_all = [x0, x1, x2, x3, x4, x5, x6, x7]
    base0 = sid * C_EDGES_PER_S

    for c in range(8):
        @pl.when(cid == c // 4)
        def _chunk(c=c):
            xc_hbm = xc_all[c]
            h = c // 2
            pltpu.sync_copy(b3_hbm.at[c], bvec.at[0])

            def fill(r, carry):
                for j in range(8):
                    bbuf[r, pl.ds(16 * j, 16)] = bvec[0, pl.ds(16 * j, 16)]
                return carry
            lax.fori_loop(0, 8, fill, 0)

            for half in range(2):
                irow0 = sid * INIT_PER_SID

                def initslab(t, carry):
                    pltpu.sync_copy(bbuf, slab.at[pl.ds(irow0 + 8 * t, 8)])
                    return carry
                lax.fori_loop(0, INIT_PER_SID // 8, initslab, 0)
                plsc.subcore_barrier()

                def blk(b, carry):
                    base = base0 + b * KB
                    pltpu.sync_copy(snd_hbm.at[pl.ds(base, KB)], sidx)
                    pltpu.sync_copy(rcv_hbm.at[pl.ds(base, KB)], ridxg)
                    pltpu.sync_copy(rcv_hbm.at[pl.ds(base, KB)], ridxs.at[0])
                    _remap_half(ridxs, half)
                    pltpu.async_copy(invd_hbm.at[ridxg], invrows, sem).wait()
                    pltpu.sync_copy(ex_hbm.at[pl.ds(base, KB)], exrows)
                    pltpu.async_copy(xc_hbm.at[sidx], xrows, sem).wait()

                    def per_edge(e, c2):
                        co = (exrows[e, pl.ds(0, 16)]
                              * invrows[e, pl.ds(0, 16)])
                        msk = lax.iota(jnp.int32, 16) == h
                        chv = _vsum16(jnp.where(msk, co, co * 0.0))
                        for j in range(8):
                            xrows[e, pl.ds(16 * j, 16)] = (
                                xrows[e, pl.ds(16 * j, 16)] * chv)
                        return c2
                    lax.fori_loop(0, KB, per_edge, 0)
                    pltpu.sync_copy(xrows, slab.at[ridxs.at[0]], add=True)
                    return carry
                lax.fori_loop(0, C_BLOCKS, blk, 0)
                plsc.subcore_barrier()

                orow0 = sid * OUT_PER_SID

                def wout(t, carry):
                    r = orow0 + 64 * t
                    pltpu.sync_copy(
                        slab.at[pl.ds(r, 64)],
                        outc_hbm.at[c, pl.ds(half * N_HALF + r, 64)])
                    return carry
                lax.fori_loop(0, OUT_PER_SID // 64, wout, 0)
                plsc.subcore_barrier()


def kernel(xs, onehots, adjs, W, att_l, att_r, conv1_w, conv1_b, conv2_w,
           conv2_b, oh_lin_w, oh_lin_b, bias):
    soh = jnp.sort(onehots, axis=-1)
    c1w = conv1_w.reshape(8, 3)
    c1b = conv1_b.reshape(1, 8)
    c2w = conv2_w.reshape(16, 24)
    c2b = conv2_b.reshape(1, 16)
    olb = oh_lin_b.reshape(1, 8)
    attl = att_l.reshape(HEADS, OUT_CH)
    attr = att_r.reshape(HEADS, OUT_CH)

    xmm, ps, pr = _dense_prep(xs, soh, onehots, c1w, c1b, c2w, c2b,
                              oh_lin_w, olb, W, attl, attr)

    adj32 = adjs.astype(jnp.int32)
    pad_idx = jnp.full((E_PAD - E,), N, jnp.int32)
    snd = jnp.concatenate([adj32[:, 0], pad_idx])
    rcv = jnp.concatenate([adj32[:, 1], pad_idx])

    ps_pad = jnp.pad(ps, ((0, N_PAD - N), (0, 0)))
    pr_pad = jnp.pad(pr, ((0, N_PAD - N), (0, 0)))

    ex, parts = _edge_pass_a_call(ps_pad, pr_pad, snd, rcv)

    oh_pad = jnp.pad(onehots, ((0, N_PAD - N), (0, 0)))
    invd, noh_pad = _combine(parts, oh_pad)

    xmm_pad = jnp.pad(xmm, ((0, N_PAD - N), (0, 0)))
    xcs = [xmm_pad[:, 128 * c:128 * (c + 1)] for c in range(8)]
    b3 = bias.reshape(8, 128)

    (outc,) = _edge_pass_c_call(*xcs, invd, ex, snd, rcv, b3)

    out = jnp.concatenate([outc[c, :N, :] for c in range(8)], axis=1)
    return out, noh_pad[:N]
